# Initial kernel scaffold; baseline (speedup 1.0000x reference)
import jax
import jax.numpy as jnp
from jax.experimental import pallas as pl


def kernel(inputs, supports, W_gl, W_msg, b_msg, Wr, Ur, Wz, Uz, Wh, Uh):
    raise NotImplementedError("write your pallas kernel here")


# speedup vs baseline: 3.0628x; 3.0628x over previous
import functools

import jax
import jax.numpy as jnp
from jax.experimental import pallas as pl

NUM_HEADS = 4
NUM_STEPS = 3
SKIP = 0.3
DH = 32


def _dotT(a, b):
    return jax.lax.dot_general(
        a, b, (((1,), (1,)), ((), ())), preferred_element_type=jnp.float32
    )


def _sglc_body(x_ref, sup_ref, wgl_ref, wmsg_ref, bmsg_ref, wr_ref, ur_ref,
               wz_ref, uz_ref, wh_ref, uh_ref, hmask_ref, out_ref, adj_ref):
    t = pl.program_id(1)
    x = x_ref[0, 0]

    xp = jnp.dot(x, wgl_ref[...], preferred_element_type=jnp.float32)
    sq = xp * xp
    s = jnp.dot(sq, hmask_ref[...], preferred_element_type=jnp.float32)
    xn = xp / (jnp.sqrt(s) + 1e-8)
    attn = jnp.maximum(_dotT(xn, xn) * (1.0 / NUM_HEADS), 0.0)
    learned = attn / (jnp.sum(attn, axis=-1, keepdims=True) + 1e-8)

    @pl.when(t == 0)
    def _init():
        adj_ref[0] = sup_ref[0]

    adj = SKIP * adj_ref[0] + (1.0 - SKIP) * learned
    adj_ref[0] = adj

    wmsg = wmsg_ref[...]
    bmsg = bmsg_ref[...]
    wr = wr_ref[...]
    ur = ur_ref[...]
    wz = wz_ref[...]
    uz = uz_ref[...]
    wh = wh_ref[...]
    uh = uh_ref[...]
    h = x
    for _ in range(NUM_STEPS):
        ah = jnp.dot(adj, h, preferred_element_type=jnp.float32)
        m = jnp.dot(ah, wmsg, preferred_element_type=jnp.float32) + bmsg
        r = jax.nn.sigmoid(
            jnp.dot(m, wr, preferred_element_type=jnp.float32)
            + jnp.dot(h, ur, preferred_element_type=jnp.float32))
        z = jax.nn.sigmoid(
            jnp.dot(m, wz, preferred_element_type=jnp.float32)
            + jnp.dot(h, uz, preferred_element_type=jnp.float32))
        hh = jnp.tanh(
            jnp.dot(m, wh, preferred_element_type=jnp.float32)
            + jnp.dot(r * h, uh, preferred_element_type=jnp.float32))
        h = (1.0 - z) * h + z * hh
    out_ref[0, 0] = h


@functools.partial(jax.jit, static_argnames=("interpret",))
def _run(inputs, supports, wglc, wmsg, bmsg, wr, ur, wz, uz, wh, uh,
         interpret=False):
    seq, b, n, d = inputs.shape
    hd = NUM_HEADS * DH
    i = jax.lax.broadcasted_iota(jnp.int32, (hd, hd), 0) // DH
    j = jax.lax.broadcasted_iota(jnp.int32, (hd, hd), 1) // DH
    hmask = (i == j).astype(jnp.float32)

    grid = (b, seq)
    out, adj = pl.pallas_call(
        _sglc_body,
        grid=grid,
        in_specs=[
            pl.BlockSpec((1, 1, n, d), lambda bi, ti: (ti, bi, 0, 0)),
            pl.BlockSpec((1, n, n), lambda bi, ti: (bi, 0, 0)),
            pl.BlockSpec((d, hd), lambda bi, ti: (0, 0)),
            pl.BlockSpec((d, d), lambda bi, ti: (0, 0)),
            pl.BlockSpec((1, d), lambda bi, ti: (0, 0)),
            pl.BlockSpec((d, d), lambda bi, ti: (0, 0)),
            pl.BlockSpec((d, d), lambda bi, ti: (0, 0)),
            pl.BlockSpec((d, d), lambda bi, ti: (0, 0)),
            pl.BlockSpec((d, d), lambda bi, ti: (0, 0)),
            pl.BlockSpec((d, d), lambda bi, ti: (0, 0)),
            pl.BlockSpec((d, d), lambda bi, ti: (0, 0)),
            pl.BlockSpec((hd, hd), lambda bi, ti: (0, 0)),
        ],
        out_specs=[
            pl.BlockSpec((1, 1, n, d), lambda bi, ti: (ti, bi, 0, 0)),
            pl.BlockSpec((1, n, n), lambda bi, ti: (bi, 0, 0)),
        ],
        out_shape=[
            jax.ShapeDtypeStruct((seq, b, n, d), jnp.float32),
            jax.ShapeDtypeStruct((b, n, n), jnp.float32),
        ],
        interpret=interpret,
    )(inputs, supports, wglc, wmsg, bmsg, wr, ur, wz, uz, wh, uh, hmask)
    return out, adj


def kernel(inputs, supports, W_gl, W_msg, b_msg, Wr, Ur, Wz, Uz, Wh, Uh):
    d = inputs.shape[-1]
    wglc = jnp.transpose(W_gl[0], (1, 0, 2)).reshape(d, NUM_HEADS * DH)
    return _run(inputs, supports, wglc, W_msg[0], b_msg[0].reshape(1, d),
                Wr[0], Ur[0], Wz[0], Uz[0], Wh[0], Uh[0])


# speedup vs baseline: 5.0237x; 1.6402x over previous
import functools

import jax
import jax.numpy as jnp
from jax.experimental import pallas as pl

NUM_HEADS = 4
NUM_STEPS = 3
SKIP = 0.3
DH = 32


def _dotT(a, b):
    return jax.lax.dot_general(
        a, b, (((1,), (1,)), ((), ())), preferred_element_type=jnp.float32
    )


def _dot(a, b):
    return jnp.dot(a, b, preferred_element_type=jnp.float32)


def _sglc_body(x_ref, sup_ref, wgl_ref, wmsg_ref, bmsg_ref, wrzh_ref,
               urz_ref, uh_ref, hmask_ref, out_ref, adj_ref):
    t = pl.program_id(0)
    nb, n, d = x_ref.shape[1], x_ref.shape[2], x_ref.shape[3]
    bf = jnp.bfloat16
    x_all = x_ref[0].reshape(nb * n, d)

    xp = _dot(x_all.astype(bf), wgl_ref[...])
    sq = (xp * xp).astype(bf)
    s = _dot(sq, hmask_ref[...])
    xn = (xp * (0.5 / (jnp.sqrt(s) + 1e-8))).astype(bf)

    @pl.when(t == 0)
    def _init():
        adj_ref[...] = sup_ref[...]

    adjs = []
    for b in range(nb):
        xnb = xn[b * n:(b + 1) * n]
        attn = jnp.maximum(_dotT(xnb, xnb), 0.0)
        learned = attn / (jnp.sum(attn, axis=-1, keepdims=True) + 1e-8)
        adj = SKIP * adj_ref[b] + (1.0 - SKIP) * learned
        adj_ref[b] = adj
        adjs.append(adj.astype(bf))

    wmsg = wmsg_ref[...]
    bmsg = bmsg_ref[...]
    wrzh = wrzh_ref[...]
    urz = urz_ref[...]
    uh = uh_ref[...]
    h = x_all
    for _ in range(NUM_STEPS):
        hb = h.astype(bf)
        a_all = jnp.concatenate(
            [_dot(adjs[b], hb[b * n:(b + 1) * n]) for b in range(nb)], axis=0)
        m = (_dot(a_all.astype(bf), wmsg) + bmsg).astype(bf)
        gates = _dot(m, wrzh)
        hu = _dot(hb, urz)
        r = jax.nn.sigmoid(gates[:, :d] + hu[:, :d])
        z = jax.nn.sigmoid(gates[:, d:2 * d] + hu[:, d:2 * d])
        q = _dot((r * h).astype(bf), uh)
        hh = jnp.tanh(gates[:, 2 * d:] + q)
        h = (1.0 - z) * h + z * hh
    out_ref[0] = h.reshape(nb, n, d)


@functools.partial(jax.jit, static_argnames=("interpret",))
def _run(inputs, supports, wglc, wmsg, bmsg, wrzh, urz, uh, interpret=False):
    seq, b, n, d = inputs.shape
    hd = NUM_HEADS * DH
    i = jax.lax.broadcasted_iota(jnp.int32, (hd, hd), 0) // DH
    j = jax.lax.broadcasted_iota(jnp.int32, (hd, hd), 1) // DH
    hmask = (i == j).astype(jnp.bfloat16)

    out, adj = pl.pallas_call(
        _sglc_body,
        grid=(seq,),
        in_specs=[
            pl.BlockSpec((1, b, n, d), lambda ti: (ti, 0, 0, 0)),
            pl.BlockSpec((b, n, n), lambda ti: (0, 0, 0)),
            pl.BlockSpec((d, hd), lambda ti: (0, 0)),
            pl.BlockSpec((d, d), lambda ti: (0, 0)),
            pl.BlockSpec((1, d), lambda ti: (0, 0)),
            pl.BlockSpec((d, 3 * d), lambda ti: (0, 0)),
            pl.BlockSpec((d, 2 * d), lambda ti: (0, 0)),
            pl.BlockSpec((d, d), lambda ti: (0, 0)),
            pl.BlockSpec((hd, hd), lambda ti: (0, 0)),
        ],
        out_specs=[
            pl.BlockSpec((1, b, n, d), lambda ti: (ti, 0, 0, 0)),
            pl.BlockSpec((b, n, n), lambda ti: (0, 0, 0)),
        ],
        out_shape=[
            jax.ShapeDtypeStruct((seq, b, n, d), jnp.float32),
            jax.ShapeDtypeStruct((b, n, n), jnp.float32),
        ],
        interpret=interpret,
    )(inputs, supports, wglc, wmsg, bmsg, wrzh, urz, uh, hmask)
    return out, adj


def kernel(inputs, supports, W_gl, W_msg, b_msg, Wr, Ur, Wz, Uz, Wh, Uh):
    d = inputs.shape[-1]
    bf = jnp.bfloat16
    wglc = jnp.transpose(W_gl[0], (1, 0, 2)).reshape(d, NUM_HEADS * DH)
    wrzh = jnp.concatenate([Wr[0], Wz[0], Wh[0]], axis=1)
    urz = jnp.concatenate([Ur[0], Uz[0]], axis=1)
    return _run(inputs, supports, wglc.astype(bf), W_msg[0].astype(bf),
                b_msg[0].reshape(1, d), wrzh.astype(bf), urz.astype(bf),
                Uh[0].astype(bf))
